# baseline (speedup 1.0000x reference)
import functools

import jax
import jax.numpy as jnp
import numpy as np
from jax import lax
from jax.experimental import pallas as pl
from jax.experimental.pallas import tpu as pltpu
from jax.experimental.pallas import tpu_sc as plsc

_NUM_BUCKETS = 100
_EMBED = 64
_HIDDEN = 128
_ROWS_PER_STEP = 2048
_NW = 32
_CHUNK = 128


def _build_body(nf_ref, nl_ref, ftp_ref, ltp_ref, wf_ref, wl_ref, b_ref,
                t3_ref, idx_ref, pf_ref, pl_ref):
    @pl.when(pl.program_id(0) == 0)
    def _():
        pf_ref[...] = jnp.dot(ftp_ref[...], wf_ref[...],
                              preferred_element_type=jnp.float32) + b_ref[...]
        pl_ref[...] = jnp.dot(ltp_ref[...], wl_ref[...],
                              preferred_element_type=jnp.float32)

    i = pl.program_id(0)
    t3_ref[0] = jnp.tanh(pf_ref[pl.ds(i, 1), :] + pl_ref[...])

    ln_day = jnp.log(jnp.float32(60 * 60 * 24.0))
    scale = jnp.float32(_NUM_BUCKETS / 7)

    def bucketize(x):
        xf = jnp.clip(x.astype(jnp.float32), 1.0, None)
        scaled = jnp.log(xf) / ln_day
        bkt = (scaled * scale).astype(jnp.int32)
        return jnp.clip(bkt, None, _NUM_BUCKETS - 1)

    fb = bucketize(nf_ref[0])
    lb = bucketize(nl_ref[0])
    idx_ref[0] = fb * _HIDDEN + lb


def _sc_gather_body(t_hbm, idx_hbm, out_hbm, idx_v, rows_v, sem):
    nc = 2
    wid = lax.axis_index("s") * nc + lax.axis_index("c")
    b_per_w = idx_hbm.shape[1] * idx_hbm.shape[2]
    nch = idx_hbm.shape[1]
    base = wid * b_per_w
    pltpu.sync_copy(idx_hbm.at[wid], idx_v)

    def chunk(c, carry):
        pltpu.async_copy(t_hbm.at[idx_v.at[c]], rows_v, sem).wait()
        pltpu.sync_copy(rows_v, out_hbm.at[pl.ds(base + c * _CHUNK, _CHUNK)])
        return carry

    lax.fori_loop(0, nch, chunk, 0)


@jax.jit
def kernel(news_freshness, news_user_topic_lifetime, freshness_table,
           lifetime_table, W, b):
    batch, news = news_freshness.shape
    n = batch * news
    steps = n // _ROWS_PER_STEP

    nf = news_freshness.reshape(steps, 1, _ROWS_PER_STEP)
    nl = news_user_topic_lifetime.reshape(steps, 1, _ROWS_PER_STEP)
    pad = jnp.zeros((_HIDDEN - _NUM_BUCKETS, _EMBED), jnp.float32)
    ftp = jnp.concatenate([freshness_table, pad], axis=0)
    ltp = jnp.concatenate([lifetime_table, pad], axis=0)
    wf = W[:, :_EMBED].T
    wl = W[:, _EMBED:].T
    b2 = b.reshape(1, _HIDDEN)

    t3, idx = pl.pallas_call(
        _build_body,
        grid=(_NUM_BUCKETS,),
        in_specs=[
            pl.BlockSpec((1, 1, _ROWS_PER_STEP), lambda i: (i, 0, 0)),
            pl.BlockSpec((1, 1, _ROWS_PER_STEP), lambda i: (i, 0, 0)),
            pl.BlockSpec((_HIDDEN, _EMBED), lambda i: (0, 0)),
            pl.BlockSpec((_HIDDEN, _EMBED), lambda i: (0, 0)),
            pl.BlockSpec((_EMBED, _HIDDEN), lambda i: (0, 0)),
            pl.BlockSpec((_EMBED, _HIDDEN), lambda i: (0, 0)),
            pl.BlockSpec((1, _HIDDEN), lambda i: (0, 0)),
        ],
        out_specs=[
            pl.BlockSpec((1, _HIDDEN, _HIDDEN), lambda i: (i, 0, 0)),
            pl.BlockSpec((1, 1, _ROWS_PER_STEP), lambda i: (i, 0, 0)),
        ],
        out_shape=[
            jax.ShapeDtypeStruct((_NUM_BUCKETS, _HIDDEN, _HIDDEN), jnp.float32),
            jax.ShapeDtypeStruct((steps, 1, _ROWS_PER_STEP), jnp.int32),
        ],
        scratch_shapes=[
            pltpu.VMEM((_HIDDEN, _HIDDEN), jnp.float32),
            pltpu.VMEM((_HIDDEN, _HIDDEN), jnp.float32),
        ],
        compiler_params=pltpu.CompilerParams(
            dimension_semantics=("arbitrary",),
        ),
    )(nf, nl, ftp, ltp, wf, wl, b2)

    t2 = t3.reshape(_NUM_BUCKETS * _HIDDEN, _HIDDEN)
    b_per_w = n // _NW
    idx3 = idx.reshape(_NW, b_per_w // _CHUNK, _CHUNK)

    mesh = plsc.VectorSubcoreMesh(core_axis_name="c", subcore_axis_name="s")
    sc = functools.partial(
        pl.kernel,
        mesh=mesh,
        out_type=jax.ShapeDtypeStruct((n, _HIDDEN), jnp.float32),
        scratch_types=[
            pltpu.VMEM((b_per_w // _CHUNK, _CHUNK), jnp.int32),
            pltpu.VMEM((_CHUNK, _HIDDEN), jnp.float32),
            pltpu.SemaphoreType.DMA,
        ],
    )(_sc_gather_body)
    out = sc(t2, idx3)
    return out.reshape(batch, news, _HIDDEN)


# speedup vs baseline: 11.8997x; 11.8997x over previous
import functools

import jax
import jax.numpy as jnp
import numpy as np
from jax.experimental import pallas as pl
from jax.experimental.pallas import tpu as pltpu

_NUM_BUCKETS = 100
_EMBED = 64
_HIDDEN = 128
_ROWS_PER_STEP = 3200


def _tc_body(nf_ref, nl_ref, ftp_ref, ltp_ref, wf_ref, wl_ref, b_ref,
             out_ref, pf_ref, pl_ref):
    @pl.when(pl.program_id(0) == 0)
    def _():
        pf_ref[...] = jnp.dot(ftp_ref[...], wf_ref[...],
                              preferred_element_type=jnp.float32) + b_ref[...]
        pl_ref[...] = jnp.dot(ltp_ref[...], wl_ref[...],
                              preferred_element_type=jnp.float32)

    ln_day = jnp.log(jnp.float32(60 * 60 * 24.0))
    scale = jnp.float32(_NUM_BUCKETS / 7)

    def bucketize(x):
        xf = jnp.clip(x.astype(jnp.float32), 1.0, None)
        scaled = jnp.log(xf) / ln_day
        bkt = (scaled * scale).astype(jnp.int32)
        return jnp.clip(bkt, None, _NUM_BUCKETS - 1)

    fb = bucketize(nf_ref[0])
    lb = bucketize(nl_ref[0])
    rows = jax.lax.broadcasted_iota(jnp.int32, (_HIDDEN, 1), 0)
    oh_f = (fb == rows).astype(jnp.float32)
    oh_l = (lb == rows).astype(jnp.float32)
    dn = (((0,), (0,)), ((), ()))
    acc = jax.lax.dot_general(oh_f, pf_ref[...], dn,
                              preferred_element_type=jnp.float32)
    acc += jax.lax.dot_general(oh_l, pl_ref[...], dn,
                               preferred_element_type=jnp.float32)
    out_ref[...] = jnp.tanh(acc).reshape(out_ref.shape)


@jax.jit
def kernel(news_freshness, news_user_topic_lifetime, freshness_table,
           lifetime_table, W, b):
    batch, news = news_freshness.shape
    n = batch * news
    steps = n // _ROWS_PER_STEP

    nf = news_freshness.reshape(steps, 1, _ROWS_PER_STEP)
    nl = news_user_topic_lifetime.reshape(steps, 1, _ROWS_PER_STEP)
    pad = jnp.zeros((_HIDDEN - _NUM_BUCKETS, _EMBED), jnp.float32)
    ftp = jnp.concatenate([freshness_table, pad], axis=0)
    ltp = jnp.concatenate([lifetime_table, pad], axis=0)
    wf = W[:, :_EMBED].T
    wl = W[:, _EMBED:].T
    b2 = b.reshape(1, _HIDDEN)

    grid = (steps,)
    out = pl.pallas_call(
        _tc_body,
        grid=grid,
        in_specs=[
            pl.BlockSpec((1, 1, _ROWS_PER_STEP), lambda i: (i, 0, 0)),
            pl.BlockSpec((1, 1, _ROWS_PER_STEP), lambda i: (i, 0, 0)),
            pl.BlockSpec((_HIDDEN, _EMBED), lambda i: (0, 0)),
            pl.BlockSpec((_HIDDEN, _EMBED), lambda i: (0, 0)),
            pl.BlockSpec((_EMBED, _HIDDEN), lambda i: (0, 0)),
            pl.BlockSpec((_EMBED, _HIDDEN), lambda i: (0, 0)),
            pl.BlockSpec((1, _HIDDEN), lambda i: (0, 0)),
        ],
        out_specs=pl.BlockSpec((_ROWS_PER_STEP // 50, news, _HIDDEN),
                               lambda i: (i, 0, 0)),
        out_shape=jax.ShapeDtypeStruct((batch, news, _HIDDEN), jnp.float32),
        scratch_shapes=[
            pltpu.VMEM((_HIDDEN, _HIDDEN), jnp.float32),
            pltpu.VMEM((_HIDDEN, _HIDDEN), jnp.float32),
        ],
        compiler_params=pltpu.CompilerParams(
            dimension_semantics=("arbitrary",),
        ),
    )(nf, nl, ftp, ltp, wf, wl, b2)
    return out


# speedup vs baseline: 12.7615x; 1.0724x over previous
import functools

import jax
import jax.numpy as jnp
import numpy as np
from jax.experimental import pallas as pl
from jax.experimental.pallas import tpu as pltpu

_NUM_BUCKETS = 100
_EMBED = 64
_HIDDEN = 128
_BATCH_PER_STEP = 64
_NEWS_PAD = 56


def _tc_body(nf_ref, nl_ref, ftp_ref, ltp_ref, wf_ref, wl_ref, b_ref,
             out_ref, pf_ref, pl_ref):
    @pl.when(pl.program_id(0) == 0)
    def _():
        pf_ref[...] = jnp.dot(ftp_ref[...], wf_ref[...],
                              preferred_element_type=jnp.float32) + b_ref[...]
        pl_ref[...] = jnp.dot(ltp_ref[...], wl_ref[...],
                              preferred_element_type=jnp.float32)

    ln_day = jnp.log(jnp.float32(60 * 60 * 24.0))
    scale = jnp.float32(_NUM_BUCKETS / 7)

    def bucketize(x):
        xf = jnp.clip(x.astype(jnp.float32), 1.0, None)
        scaled = jnp.log(xf) / ln_day
        bkt = (scaled * scale).astype(jnp.int32)
        return jnp.clip(bkt, None, _NUM_BUCKETS - 1)

    fb = bucketize(nf_ref[0])
    lb = bucketize(nl_ref[0])
    rows = jax.lax.broadcasted_iota(jnp.int32, (_HIDDEN, 1), 0)
    oh_f = (fb == rows).astype(jnp.float32)
    oh_l = (lb == rows).astype(jnp.float32)
    dn = (((0,), (0,)), ((), ()))
    acc = jax.lax.dot_general(oh_f, pf_ref[...], dn,
                              preferred_element_type=jnp.float32)
    acc += jax.lax.dot_general(oh_l, pl_ref[...], dn,
                               preferred_element_type=jnp.float32)
    acc3 = acc.reshape(_BATCH_PER_STEP, _NEWS_PAD, _HIDDEN)
    out_ref[...] = jnp.tanh(acc3[:, :out_ref.shape[1], :])


@jax.jit
def kernel(news_freshness, news_user_topic_lifetime, freshness_table,
           lifetime_table, W, b):
    batch, news = news_freshness.shape
    steps = batch // _BATCH_PER_STEP
    epb = _BATCH_PER_STEP * _NEWS_PAD

    def prep(x):
        xp = jnp.pad(x, ((0, 0), (0, _NEWS_PAD - news)), constant_values=1)
        return xp.reshape(steps, 1, epb)

    nf = prep(news_freshness)
    nl = prep(news_user_topic_lifetime)
    pad = jnp.zeros((_HIDDEN - _NUM_BUCKETS, _EMBED), jnp.float32)
    ftp = jnp.concatenate([freshness_table, pad], axis=0)
    ltp = jnp.concatenate([lifetime_table, pad], axis=0)
    wf = W[:, :_EMBED].T
    wl = W[:, _EMBED:].T
    b2 = b.reshape(1, _HIDDEN)

    out = pl.pallas_call(
        _tc_body,
        grid=(steps,),
        in_specs=[
            pl.BlockSpec((1, 1, epb), lambda i: (i, 0, 0)),
            pl.BlockSpec((1, 1, epb), lambda i: (i, 0, 0)),
            pl.BlockSpec((_HIDDEN, _EMBED), lambda i: (0, 0)),
            pl.BlockSpec((_HIDDEN, _EMBED), lambda i: (0, 0)),
            pl.BlockSpec((_EMBED, _HIDDEN), lambda i: (0, 0)),
            pl.BlockSpec((_EMBED, _HIDDEN), lambda i: (0, 0)),
            pl.BlockSpec((1, _HIDDEN), lambda i: (0, 0)),
        ],
        out_specs=pl.BlockSpec((_BATCH_PER_STEP, news, _HIDDEN),
                               lambda i: (i, 0, 0)),
        out_shape=jax.ShapeDtypeStruct((batch, news, _HIDDEN), jnp.float32),
        scratch_shapes=[
            pltpu.VMEM((_HIDDEN, _HIDDEN), jnp.float32),
            pltpu.VMEM((_HIDDEN, _HIDDEN), jnp.float32),
        ],
        compiler_params=pltpu.CompilerParams(
            dimension_semantics=("arbitrary",),
        ),
    )(nf, nl, ftp, ltp, wf, wl, b2)
    return out


# speedup vs baseline: 14.0696x; 1.1025x over previous
import functools

import jax
import jax.numpy as jnp
import numpy as np
from jax.experimental import pallas as pl
from jax.experimental.pallas import tpu as pltpu

_NUM_BUCKETS = 100
_EMBED = 64
_HIDDEN = 128
_BATCH_PER_STEP = 128
_NEWS_PAD = 56


def _tc_body(nf_ref, nl_ref, ftp_ref, ltp_ref, wf_ref, wl_ref, b_ref,
             out_ref, pf_ref, pl_ref):
    @pl.when(pl.program_id(0) == 0)
    def _():
        pf_ref[...] = jnp.dot(ftp_ref[...], wf_ref[...],
                              preferred_element_type=jnp.float32) + b_ref[...]
        pl_ref[...] = jnp.dot(ltp_ref[...], wl_ref[...],
                              preferred_element_type=jnp.float32)

    ln_day = jnp.log(jnp.float32(60 * 60 * 24.0))
    scale = jnp.float32(_NUM_BUCKETS / 7)

    def bucketize(x):
        xf = jnp.clip(x.astype(jnp.float32), 1.0, None)
        scaled = jnp.log(xf) / ln_day
        bkt = (scaled * scale).astype(jnp.int32)
        return jnp.clip(bkt, None, _NUM_BUCKETS - 1)

    fb = bucketize(nf_ref[0])
    lb = bucketize(nl_ref[0])
    rows = jax.lax.broadcasted_iota(jnp.int32, (_HIDDEN, 1), 0)
    oh_f = (fb == rows).astype(jnp.float32)
    oh_l = (lb == rows).astype(jnp.float32)
    dn = (((0,), (0,)), ((), ()))
    acc = jax.lax.dot_general(oh_f, pf_ref[...], dn,
                              preferred_element_type=jnp.float32)
    acc += jax.lax.dot_general(oh_l, pl_ref[...], dn,
                               preferred_element_type=jnp.float32)
    acc3 = acc.reshape(_BATCH_PER_STEP, _NEWS_PAD, _HIDDEN)
    out_ref[...] = jnp.tanh(acc3[:, :out_ref.shape[1], :])


@jax.jit
def kernel(news_freshness, news_user_topic_lifetime, freshness_table,
           lifetime_table, W, b):
    batch, news = news_freshness.shape
    steps = batch // _BATCH_PER_STEP
    epb = _BATCH_PER_STEP * _NEWS_PAD

    def prep(x):
        xp = jnp.pad(x, ((0, 0), (0, _NEWS_PAD - news)), constant_values=1)
        return xp.reshape(steps, 1, epb)

    nf = prep(news_freshness)
    nl = prep(news_user_topic_lifetime)
    pad = jnp.zeros((_HIDDEN - _NUM_BUCKETS, _EMBED), jnp.float32)
    ftp = jnp.concatenate([freshness_table, pad], axis=0)
    ltp = jnp.concatenate([lifetime_table, pad], axis=0)
    wf = W[:, :_EMBED].T
    wl = W[:, _EMBED:].T
    b2 = b.reshape(1, _HIDDEN)

    out = pl.pallas_call(
        _tc_body,
        grid=(steps,),
        in_specs=[
            pl.BlockSpec((1, 1, epb), lambda i: (i, 0, 0)),
            pl.BlockSpec((1, 1, epb), lambda i: (i, 0, 0)),
            pl.BlockSpec((_HIDDEN, _EMBED), lambda i: (0, 0)),
            pl.BlockSpec((_HIDDEN, _EMBED), lambda i: (0, 0)),
            pl.BlockSpec((_EMBED, _HIDDEN), lambda i: (0, 0)),
            pl.BlockSpec((_EMBED, _HIDDEN), lambda i: (0, 0)),
            pl.BlockSpec((1, _HIDDEN), lambda i: (0, 0)),
        ],
        out_specs=pl.BlockSpec((_BATCH_PER_STEP, news, _HIDDEN),
                               lambda i: (i, 0, 0)),
        out_shape=jax.ShapeDtypeStruct((batch, news, _HIDDEN), jnp.float32),
        scratch_shapes=[
            pltpu.VMEM((_HIDDEN, _HIDDEN), jnp.float32),
            pltpu.VMEM((_HIDDEN, _HIDDEN), jnp.float32),
        ],
        compiler_params=pltpu.CompilerParams(
            dimension_semantics=("arbitrary",),
        ),
    )(nf, nl, ftp, ltp, wf, wl, b2)
    return out


# speedup vs baseline: 14.3230x; 1.0180x over previous
import functools

import jax
import jax.numpy as jnp
import numpy as np
from jax.experimental import pallas as pl
from jax.experimental.pallas import tpu as pltpu

_NUM_BUCKETS = 100
_EMBED = 64
_HIDDEN = 128
_BATCH_PER_STEP = 256
_NEWS_PAD = 56


def _tc_body(nf_ref, nl_ref, ftp_ref, ltp_ref, wf_ref, wl_ref, b_ref,
             out_ref, pf_ref, pl_ref):
    @pl.when(pl.program_id(0) == 0)
    def _():
        pf_ref[...] = jnp.dot(ftp_ref[...], wf_ref[...],
                              preferred_element_type=jnp.float32) + b_ref[...]
        pl_ref[...] = jnp.dot(ltp_ref[...], wl_ref[...],
                              preferred_element_type=jnp.float32)

    ln_day = jnp.log(jnp.float32(60 * 60 * 24.0))
    scale = jnp.float32(_NUM_BUCKETS / 7)

    def bucketize(x):
        xf = jnp.clip(x.astype(jnp.float32), 1.0, None)
        scaled = jnp.log(xf) / ln_day
        bkt = (scaled * scale).astype(jnp.int32)
        return jnp.clip(bkt, None, _NUM_BUCKETS - 1)

    fb = bucketize(nf_ref[0])
    lb = bucketize(nl_ref[0])
    rows = jax.lax.broadcasted_iota(jnp.int32, (_HIDDEN, 1), 0)
    oh_f = (fb == rows).astype(jnp.float32)
    oh_l = (lb == rows).astype(jnp.float32)
    dn = (((0,), (0,)), ((), ()))
    acc = jax.lax.dot_general(oh_f, pf_ref[...], dn,
                              preferred_element_type=jnp.float32)
    acc += jax.lax.dot_general(oh_l, pl_ref[...], dn,
                               preferred_element_type=jnp.float32)
    acc3 = acc.reshape(_BATCH_PER_STEP, _NEWS_PAD, _HIDDEN)
    out_ref[...] = jnp.tanh(acc3[:, :out_ref.shape[1], :])


@jax.jit
def kernel(news_freshness, news_user_topic_lifetime, freshness_table,
           lifetime_table, W, b):
    batch, news = news_freshness.shape
    steps = batch // _BATCH_PER_STEP
    epb = _BATCH_PER_STEP * _NEWS_PAD

    def prep(x):
        xp = jnp.pad(x, ((0, 0), (0, _NEWS_PAD - news)), constant_values=1)
        return xp.reshape(steps, 1, epb)

    nf = prep(news_freshness)
    nl = prep(news_user_topic_lifetime)
    pad = jnp.zeros((_HIDDEN - _NUM_BUCKETS, _EMBED), jnp.float32)
    ftp = jnp.concatenate([freshness_table, pad], axis=0)
    ltp = jnp.concatenate([lifetime_table, pad], axis=0)
    wf = W[:, :_EMBED].T
    wl = W[:, _EMBED:].T
    b2 = b.reshape(1, _HIDDEN)

    out = pl.pallas_call(
        _tc_body,
        grid=(steps,),
        in_specs=[
            pl.BlockSpec((1, 1, epb), lambda i: (i, 0, 0)),
            pl.BlockSpec((1, 1, epb), lambda i: (i, 0, 0)),
            pl.BlockSpec((_HIDDEN, _EMBED), lambda i: (0, 0)),
            pl.BlockSpec((_HIDDEN, _EMBED), lambda i: (0, 0)),
            pl.BlockSpec((_EMBED, _HIDDEN), lambda i: (0, 0)),
            pl.BlockSpec((_EMBED, _HIDDEN), lambda i: (0, 0)),
            pl.BlockSpec((1, _HIDDEN), lambda i: (0, 0)),
        ],
        out_specs=pl.BlockSpec((_BATCH_PER_STEP, news, _HIDDEN),
                               lambda i: (i, 0, 0)),
        out_shape=jax.ShapeDtypeStruct((batch, news, _HIDDEN), jnp.float32),
        scratch_shapes=[
            pltpu.VMEM((_HIDDEN, _HIDDEN), jnp.float32),
            pltpu.VMEM((_HIDDEN, _HIDDEN), jnp.float32),
        ],
        compiler_params=pltpu.CompilerParams(
            dimension_semantics=("arbitrary",),
        ),
    )(nf, nl, ftp, ltp, wf, wl, b2)
    return out
